# Initial kernel scaffold; baseline (speedup 1.0000x reference)
import jax
import jax.numpy as jnp
from jax.experimental import pallas as pl


def kernel(x, edge_index, batch, W1, b1, W2, b2):
    raise NotImplementedError("write your pallas kernel here")


# speedup vs baseline: 7.9723x; 7.9723x over previous
import functools

import jax
import jax.numpy as jnp
from jax import lax
from jax.experimental import pallas as pl
from jax.experimental.pallas import tpu as pltpu
from jax.experimental.pallas import tpu_sc as plsc

N = 10000
E = 160000
D_IN, D1, D2 = 128, 64, 32

NC, NS = 2, 16
NW = NC * NS
CHUNK = 128
EPT = 5120
E_PAD = NW * EPT
NCH = EPT // CHUNK
NPAD = 10240
RPT = NPAD // NS

_mesh = plsc.VectorSubcoreMesh(core_axis_name="c", subcore_axis_name="s")


def _wid(c, s):
    return s * NC + c


@functools.partial(
    pl.kernel,
    mesh=_mesh,
    out_type=jax.ShapeDtypeStruct((NC, NPAD), jnp.float32),
    compiler_params=pltpu.CompilerParams(use_tc_tiling_on_sc=False),
    scratch_types=[
        pltpu.VMEM((CHUNK,), jnp.int32),
        pltpu.VMEM((CHUNK,), jnp.float32),
        pltpu.VMEM_SHARED((NPAD,), jnp.float32),
    ],
)
def _deg_kernel(dst_hbm, zvec_hbm, out_hbm, dst_v, ones_v, acc):
    c = lax.axis_index("c")
    s = lax.axis_index("s")
    w = _wid(c, s)
    rbase = s * RPT
    for i in range(CHUNK // 16):
        ones_v[pl.ds(i * 16, 16)] = jnp.ones((16,), jnp.float32)
    pltpu.sync_copy(zvec_hbm.at[pl.ds(rbase, RPT)], acc.at[pl.ds(rbase, RPT)])
    plsc.subcore_barrier()

    def body(j, carry):
        base = w * EPT + j * CHUNK
        pltpu.sync_copy(dst_hbm.at[pl.ds(base, CHUNK)], dst_v)
        pltpu.sync_copy(ones_v, acc.at[dst_v], add=True)
        return carry

    lax.fori_loop(0, NCH, body, 0)
    plsc.subcore_barrier()
    pltpu.sync_copy(acc.at[pl.ds(rbase, RPT)], out_hbm.at[c, pl.ds(rbase, RPT)])


def _make_agg(D):
    @functools.partial(
        pl.kernel,
        mesh=_mesh,
        out_type=jax.ShapeDtypeStruct((NC, NPAD, D), jnp.float32),
        compiler_params=pltpu.CompilerParams(use_tc_tiling_on_sc=False),
        scratch_types=[
            pltpu.VMEM((CHUNK,), jnp.int32),
            pltpu.VMEM((CHUNK,), jnp.int32),
            pltpu.VMEM((CHUNK, D), jnp.float32),
            pltpu.VMEM_SHARED((NPAD, D), jnp.float32),
            pltpu.SemaphoreType.DMA,
        ],
    )
    def _agg(g_hbm, src_hbm, dst_hbm, zmat_hbm, out_hbm, src_v, dst_v, rows_v,
             acc, sem):
        c = lax.axis_index("c")
        s = lax.axis_index("s")
        w = _wid(c, s)
        rbase = s * RPT
        pltpu.sync_copy(zmat_hbm.at[pl.ds(rbase, RPT)], acc.at[pl.ds(rbase, RPT)])
        plsc.subcore_barrier()

        def body(j, carry):
            base = w * EPT + j * CHUNK
            pltpu.sync_copy(src_hbm.at[pl.ds(base, CHUNK)], src_v)
            pltpu.sync_copy(dst_hbm.at[pl.ds(base, CHUNK)], dst_v)
            pltpu.async_copy(g_hbm.at[src_v], rows_v, sem).wait()
            pltpu.sync_copy(rows_v, acc.at[dst_v], add=True)
            return carry

        lax.fori_loop(0, NCH, body, 0)
        plsc.subcore_barrier()
        pltpu.sync_copy(acc.at[pl.ds(rbase, RPT)],
                        out_hbm.at[c, pl.ds(rbase, RPT)])

    return _agg


_agg64 = _make_agg(D1)
_agg32 = _make_agg(D2)


def _mm1_body(x_ref, w_ref, o_ref):
    o_ref[...] = jnp.dot(x_ref[...], w_ref[...],
                         preferred_element_type=jnp.float32)


def _mm1(x, W1):
    return pl.pallas_call(
        _mm1_body,
        out_shape=jax.ShapeDtypeStruct((N, D1), jnp.float32),
    )(x, W1)


def _scale_body(h_ref, d0_ref, d1_ref, o_ref):
    dinv = lax.rsqrt(d0_ref[...] + d1_ref[...] + 1.0)
    o_ref[...] = h_ref[...] * dinv


def _scale(h, d0, d1):
    return pl.pallas_call(
        _scale_body,
        out_shape=jax.ShapeDtypeStruct(h.shape, jnp.float32),
    )(h, d0, d1)


def _l1l2_body(a0_ref, a1_ref, g1_ref, d0_ref, d1_ref, b1_ref, w2_ref, o_ref):
    dinv = lax.rsqrt(d0_ref[...] + d1_ref[...] + 1.0)
    pre = dinv * (a0_ref[...] + a1_ref[...] + g1_ref[...]) + b1_ref[...]
    o1 = jnp.where(pre >= 0, pre, 0.01 * pre)
    o_ref[...] = jnp.dot(o1, w2_ref[...],
                         preferred_element_type=jnp.float32) * dinv


def _l1l2(a0, a1, g1, d0, d1, b1, W2):
    return pl.pallas_call(
        _l1l2_body,
        out_shape=jax.ShapeDtypeStruct((N, D2), jnp.float32),
    )(a0, a1, g1, d0, d1, b1, W2)


def _fin_body(a0_ref, a1_ref, g2_ref, d0_ref, d1_ref, b2_ref, o_ref):
    dinv = lax.rsqrt(d0_ref[...] + d1_ref[...] + 1.0)
    pre = dinv * (a0_ref[...] + a1_ref[...] + g2_ref[...]) + b2_ref[...]
    o_ref[...] = jnp.where(pre >= 0, pre, 0.01 * pre)


def _fin(a0, a1, g2, d0, d1, b2):
    return pl.pallas_call(
        _fin_body,
        out_shape=jax.ShapeDtypeStruct((N, D2), jnp.float32),
    )(a0, a1, g2, d0, d1, b2)


_BM, _BN = 512, 2048


def _dec_body(hl_ref, hr_ref, o_ref):
    z = lax.dot_general(hl_ref[...], hr_ref[...],
                        (((1,), (1,)), ((), ())),
                        preferred_element_type=jnp.float32)
    o_ref[...] = 1.0 / (1.0 + jnp.exp(-z))


def _decoder(h):
    gm = (N + _BM - 1) // _BM
    gn = (N + _BN - 1) // _BN
    return pl.pallas_call(
        _dec_body,
        grid=(gm, gn),
        in_specs=[
            pl.BlockSpec((_BM, D2), lambda i, j: (i, 0)),
            pl.BlockSpec((_BN, D2), lambda i, j: (j, 0)),
        ],
        out_specs=pl.BlockSpec((_BM, _BN), lambda i, j: (i, j)),
        out_shape=jax.ShapeDtypeStruct((N, N), jnp.float32),
    )(h, h)


def kernel(x, edge_index, batch, W1, b1, W2, b2):
    del batch
    src = edge_index[0].astype(jnp.int32)
    dst = edge_index[1].astype(jnp.int32)
    pad = E_PAD - E
    src_p = jnp.concatenate([src, jnp.zeros((pad,), jnp.int32)])
    dst_p = jnp.concatenate([dst, jnp.full((pad,), N, jnp.int32)])

    zvec = jnp.zeros((NPAD,), jnp.float32)
    zmat1 = jnp.zeros((NPAD, D1), jnp.float32)
    zmat2 = jnp.zeros((NPAD, D2), jnp.float32)

    degp = _deg_kernel(dst_p, zvec)
    d0 = degp[0, :N].reshape(N, 1)
    d1 = degp[1, :N].reshape(N, 1)

    h1 = _mm1(x, W1)
    g1 = _scale(h1, d0, d1)

    aggp1 = _agg64(g1, src_p, dst_p, zmat1)
    g2 = _l1l2(aggp1[0, :N], aggp1[1, :N], g1, d0, d1,
               b1.reshape(1, D1), W2)

    aggp2 = _agg32(g2, src_p, dst_p, zmat2)
    h = _fin(aggp2[0, :N], aggp2[1, :N], g2, d0, d1, b2.reshape(1, D2))

    x1 = _decoder(h)
    return (x1, h)


# speedup vs baseline: 10.2441x; 1.2850x over previous
import functools

import jax
import jax.numpy as jnp
from jax import lax
from jax.experimental import pallas as pl
from jax.experimental.pallas import tpu as pltpu
from jax.experimental.pallas import tpu_sc as plsc

N = 10000
E = 160000
D_IN, D1, D2 = 128, 64, 32

NC, NS = 2, 16
NW = NC * NS
CHUNK = 128
EPT = 5120
E_PAD = NW * EPT
NCH = EPT // CHUNK
NPAD = 10240
RPT = NPAD // NS

_mesh = plsc.VectorSubcoreMesh(core_axis_name="c", subcore_axis_name="s")


def _wid(c, s):
    return s * NC + c


@functools.partial(
    pl.kernel,
    mesh=_mesh,
    out_type=jax.ShapeDtypeStruct((NC, NPAD), jnp.float32),
    compiler_params=pltpu.CompilerParams(use_tc_tiling_on_sc=False),
    scratch_types=[
        pltpu.VMEM((NCH, CHUNK), jnp.int32),
        pltpu.VMEM((CHUNK,), jnp.float32),
        pltpu.VMEM_SHARED((NPAD,), jnp.float32),
        pltpu.SemaphoreType.DMA,
    ],
)
def _deg_kernel(dst_hbm, zvec_hbm, out_hbm, dst_v, ones_v, acc, sem):
    c = lax.axis_index("c")
    s = lax.axis_index("s")
    w = _wid(c, s)
    rbase = s * RPT
    for i in range(CHUNK // 16):
        ones_v[pl.ds(i * 16, 16)] = jnp.ones((16,), jnp.float32)
    pltpu.sync_copy(dst_hbm.at[w], dst_v)
    pltpu.sync_copy(zvec_hbm.at[pl.ds(rbase, RPT)], acc.at[pl.ds(rbase, RPT)])
    plsc.subcore_barrier()

    def fire(j, carry):
        pltpu.async_copy(ones_v, acc.at[dst_v.at[j]], sem, add=True)
        return carry

    lax.fori_loop(0, NCH, fire, 0)

    def drain(j, carry):
        pltpu.make_async_copy(ones_v, acc.at[dst_v.at[0]], sem).wait()
        return carry

    lax.fori_loop(0, NCH, drain, 0)
    plsc.subcore_barrier()
    pltpu.sync_copy(acc.at[pl.ds(rbase, RPT)], out_hbm.at[c, pl.ds(rbase, RPT)])


def _make_agg(D):
    NBUF = 4

    @functools.partial(
        pl.kernel,
        mesh=_mesh,
        out_type=jax.ShapeDtypeStruct((NC, NPAD, D), jnp.float32),
        compiler_params=pltpu.CompilerParams(use_tc_tiling_on_sc=False),
        scratch_types=(
            [pltpu.VMEM((NCH, CHUNK), jnp.int32),
             pltpu.VMEM((NCH, CHUNK), jnp.int32)]
            + [pltpu.VMEM((CHUNK, D), jnp.float32)] * NBUF
            + [pltpu.VMEM_SHARED((NPAD, D), jnp.float32)]
            + [pltpu.SemaphoreType.DMA] * (2 * NBUF)
        ),
    )
    def _agg(g_hbm, src_hbm, dst_hbm, zmat_hbm, out_hbm, src_v, dst_v,
             *rest):
        rows = rest[:NBUF]
        acc = rest[NBUF]
        sem_g = rest[NBUF + 1:NBUF + 1 + NBUF]
        sem_s = rest[NBUF + 1 + NBUF:]
        c = lax.axis_index("c")
        s = lax.axis_index("s")
        w = _wid(c, s)
        rbase = s * RPT
        pltpu.sync_copy(src_hbm.at[w], src_v)
        pltpu.sync_copy(dst_hbm.at[w], dst_v)
        pltpu.sync_copy(zmat_hbm.at[pl.ds(rbase, RPT)], acc.at[pl.ds(rbase, RPT)])
        plsc.subcore_barrier()

        for b in range(NBUF):
            pltpu.async_copy(g_hbm.at[src_v.at[b]], rows[b], sem_g[b])

        def group(gi, carry):
            for b in range(NBUF):
                j = gi * NBUF + b
                pltpu.make_async_copy(g_hbm.at[pl.ds(0, CHUNK)], rows[b],
                                      sem_g[b]).wait()
                pltpu.async_copy(rows[b], acc.at[dst_v.at[j]], sem_s[b],
                                 add=True)
            for b in range(NBUF):
                j = gi * NBUF + b
                pltpu.make_async_copy(g_hbm.at[pl.ds(0, CHUNK)], rows[b],
                                      sem_s[b]).wait()
                nj = j + NBUF

                @pl.when(nj < NCH)
                def _():
                    pltpu.async_copy(g_hbm.at[src_v.at[nj]], rows[b],
                                     sem_g[b])
            return carry

        lax.fori_loop(0, NCH // NBUF, group, 0)
        plsc.subcore_barrier()
        pltpu.sync_copy(acc.at[pl.ds(rbase, RPT)],
                        out_hbm.at[c, pl.ds(rbase, RPT)])

    return _agg


_agg64 = _make_agg(D1)
_agg32 = _make_agg(D2)


def _mm1_body(x_ref, w_ref, o_ref):
    o_ref[...] = jnp.dot(x_ref[...], w_ref[...],
                         preferred_element_type=jnp.float32)


def _mm1(x, W1):
    return pl.pallas_call(
        _mm1_body,
        out_shape=jax.ShapeDtypeStruct((N, D1), jnp.float32),
    )(x, W1)


def _scale_body(h_ref, d0_ref, d1_ref, o_ref):
    dinv = lax.rsqrt(d0_ref[...] + d1_ref[...] + 1.0)
    o_ref[...] = h_ref[...] * dinv


def _scale(h, d0, d1):
    return pl.pallas_call(
        _scale_body,
        out_shape=jax.ShapeDtypeStruct(h.shape, jnp.float32),
    )(h, d0, d1)


def _l1l2_body(a0_ref, a1_ref, g1_ref, d0_ref, d1_ref, b1_ref, w2_ref, o_ref):
    dinv = lax.rsqrt(d0_ref[...] + d1_ref[...] + 1.0)
    pre = dinv * (a0_ref[...] + a1_ref[...] + g1_ref[...]) + b1_ref[...]
    o1 = jnp.where(pre >= 0, pre, 0.01 * pre)
    o_ref[...] = jnp.dot(o1, w2_ref[...],
                         preferred_element_type=jnp.float32) * dinv


def _l1l2(a0, a1, g1, d0, d1, b1, W2):
    return pl.pallas_call(
        _l1l2_body,
        out_shape=jax.ShapeDtypeStruct((N, D2), jnp.float32),
    )(a0, a1, g1, d0, d1, b1, W2)


def _fin_body(a0_ref, a1_ref, g2_ref, d0_ref, d1_ref, b2_ref, o_ref):
    dinv = lax.rsqrt(d0_ref[...] + d1_ref[...] + 1.0)
    pre = dinv * (a0_ref[...] + a1_ref[...] + g2_ref[...]) + b2_ref[...]
    o_ref[...] = jnp.where(pre >= 0, pre, 0.01 * pre)


def _fin(a0, a1, g2, d0, d1, b2):
    return pl.pallas_call(
        _fin_body,
        out_shape=jax.ShapeDtypeStruct((N, D2), jnp.float32),
    )(a0, a1, g2, d0, d1, b2)


_BM, _BN = 512, 2048


def _dec_body(hl_ref, hr_ref, o_ref):
    z = lax.dot_general(hl_ref[...], hr_ref[...],
                        (((1,), (1,)), ((), ())),
                        preferred_element_type=jnp.float32)
    o_ref[...] = 0.5 + 0.5 * jnp.tanh(0.5 * z)


def _decoder(h):
    gm = (N + _BM - 1) // _BM
    gn = (N + _BN - 1) // _BN
    return pl.pallas_call(
        _dec_body,
        grid=(gm, gn),
        in_specs=[
            pl.BlockSpec((_BM, D2), lambda i, j: (i, 0)),
            pl.BlockSpec((_BN, D2), lambda i, j: (j, 0)),
        ],
        out_specs=pl.BlockSpec((_BM, _BN), lambda i, j: (i, j)),
        out_shape=jax.ShapeDtypeStruct((N, N), jnp.float32),
    )(h, h)


def kernel(x, edge_index, batch, W1, b1, W2, b2):
    del batch
    src = edge_index[0].astype(jnp.int32)
    dst = edge_index[1].astype(jnp.int32)
    pad = E_PAD - E
    src_p = jnp.concatenate([src, jnp.zeros((pad,), jnp.int32)])
    dst_p = jnp.concatenate([dst, jnp.full((pad,), N, jnp.int32)])
    src_p = src_p.reshape(NW, NCH, CHUNK)
    dst_p = dst_p.reshape(NW, NCH, CHUNK)

    zvec = jnp.zeros((NPAD,), jnp.float32)
    zmat1 = jnp.zeros((NPAD, D1), jnp.float32)
    zmat2 = jnp.zeros((NPAD, D2), jnp.float32)

    degp = _deg_kernel(dst_p, zvec)
    d0 = degp[0, :N].reshape(N, 1)
    d1 = degp[1, :N].reshape(N, 1)

    h1 = _mm1(x, W1)
    g1 = _scale(h1, d0, d1)

    aggp1 = _agg64(g1, src_p, dst_p, zmat1)
    g2 = _l1l2(aggp1[0, :N], aggp1[1, :N], g1, d0, d1,
               b1.reshape(1, D1), W2)

    aggp2 = _agg32(g2, src_p, dst_p, zmat2)
    h = _fin(aggp2[0, :N], aggp2[1, :N], g2, d0, d1, b2.reshape(1, D2))

    x1 = _decoder(h)
    return (x1, h)
